# Initial kernel scaffold; baseline (speedup 1.0000x reference)
import jax
import jax.numpy as jnp
from jax.experimental import pallas as pl


def kernel(x, edge_index, edge_attr, W_in, b_in, ln1_g, ln1_b, W_msg, b_msg, W_upd, b_upd, ln2_g, ln2_b):
    raise NotImplementedError("write your pallas kernel here")


# speedup vs baseline: 1.8593x; 1.8593x over previous
import jax
import jax.numpy as jnp
from jax import lax
from jax.experimental import pallas as pl
from jax.experimental.pallas import tpu as pltpu
from jax.experimental.pallas import tpu_sc as plsc

N = 50000
E = 800000
NODE_DIM = 7
EDGE_DIM = 5
H = 64
HH = H // 2

NP = 50048
STRIPE = NP // 16
TILE_EP = NP
E_PAD = 16 * TILE_EP
SB = 128
NSB = TILE_EP // SB

_C = 0.7978845608028654
_K1 = -2.0 * _C
_K2 = -2.0 * _C * 0.044715


def _ln(x, g, b):
    m = jnp.mean(x, axis=-1, keepdims=True)
    v = jnp.mean((x - m) ** 2, axis=-1, keepdims=True)
    return (x - m) * jax.lax.rsqrt(v + 1e-5) * g + b


_NODE_BLK = 3128
_EDGE_BLK4 = 3128


def _pre_body(x_ref, w_in_ref, b_in_ref, g1_ref, b1_ref, w1_ref, w2_ref,
              h_ref, a_ref, b_ref):
    h = jnp.dot(x_ref[...], w_in_ref[...], preferred_element_type=jnp.float32)
    h = h + b_in_ref[...]
    h = _ln(h, g1_ref[...], b1_ref[...])
    h = 0.5 * h * (1.0 + lax.erf(h * 0.7071067811865476))
    h_ref[...] = h
    a = jnp.dot(h, w1_ref[...], preferred_element_type=jnp.float32)
    b = jnp.dot(h, w2_ref[...], preferred_element_type=jnp.float32)
    a_ref[0] = a[:, :HH]
    a_ref[1] = a[:, HH:]
    b_ref[0] = b[:, :HH]
    b_ref[1] = b[:, HH:]


def _tc_pre(x_pad, W_in, b_in, g1, b1, W1, W2):
    grid = (NP // _NODE_BLK,)
    return pl.pallas_call(
        _pre_body,
        grid=grid,
        in_specs=[
            pl.BlockSpec((_NODE_BLK, NODE_DIM), lambda i: (i, 0)),
            pl.BlockSpec((NODE_DIM, H), lambda i: (0, 0)),
            pl.BlockSpec((1, H), lambda i: (0, 0)),
            pl.BlockSpec((1, H), lambda i: (0, 0)),
            pl.BlockSpec((1, H), lambda i: (0, 0)),
            pl.BlockSpec((H, H), lambda i: (0, 0)),
            pl.BlockSpec((H, H), lambda i: (0, 0)),
        ],
        out_specs=[
            pl.BlockSpec((_NODE_BLK, H), lambda i: (i, 0)),
            pl.BlockSpec((2, _NODE_BLK, HH), lambda i: (0, i, 0)),
            pl.BlockSpec((2, _NODE_BLK, HH), lambda i: (0, i, 0)),
        ],
        out_shape=[
            jax.ShapeDtypeStruct((NP, H), jnp.float32),
            jax.ShapeDtypeStruct((2, NP, HH), jnp.float32),
            jax.ShapeDtypeStruct((2, NP, HH), jnp.float32),
        ],
    )(x_pad, W_in, b_in, g1, b1, W1, W2)


def _cproj_body(ea4_ref, w0_ref, w1_ref, b0_ref, b1_ref, c_ref):
    ea4 = ea4_ref[...]
    c_ref[0] = jnp.dot(ea4, w0_ref[...],
                       preferred_element_type=jnp.float32) + b0_ref[...]
    c_ref[1] = jnp.dot(ea4, w1_ref[...],
                       preferred_element_type=jnp.float32) + b1_ref[...]


def _tc_cproj(ea4, W3_0, W3_1, b4_0, b4_1):
    grid = (E_PAD // 4 // _EDGE_BLK4,)
    return pl.pallas_call(
        _cproj_body,
        grid=grid,
        in_specs=[
            pl.BlockSpec((_EDGE_BLK4, 4 * EDGE_DIM), lambda i: (i, 0)),
            pl.BlockSpec((4 * EDGE_DIM, 4 * HH), lambda i: (0, 0)),
            pl.BlockSpec((4 * EDGE_DIM, 4 * HH), lambda i: (0, 0)),
            pl.BlockSpec((1, 4 * HH), lambda i: (0, 0)),
            pl.BlockSpec((1, 4 * HH), lambda i: (0, 0)),
        ],
        out_specs=pl.BlockSpec((2, _EDGE_BLK4, 4 * HH), lambda i: (0, i, 0)),
        out_shape=jax.ShapeDtypeStruct((2, E_PAD // 4, 4 * HH), jnp.float32),
    )(ea4, W3_0, W3_1, b4_0, b4_1)


def _post_body(h_ref, g0_ref, g1_ref, wu_ref, bu_ref, ln_g_ref, ln_b_ref,
               o_ref):
    wu = wu_ref[...]
    upd = jnp.dot(g0_ref[0], wu[:HH, :], preferred_element_type=jnp.float32)
    upd = upd + jnp.dot(g1_ref[0], wu[HH:, :],
                        preferred_element_type=jnp.float32)
    upd = upd + bu_ref[...]
    o_ref[...] = _ln(h_ref[...] + upd, ln_g_ref[...], ln_b_ref[...])


def _tc_post(h_pad, agg, W_upd, b_upd, g2, b2):
    grid = (N // 2000,)
    return pl.pallas_call(
        _post_body,
        grid=grid,
        in_specs=[
            pl.BlockSpec((2000, H), lambda i: (i, 0)),
            pl.BlockSpec((1, 2000, HH), lambda i: (0, i, 0)),
            pl.BlockSpec((1, 2000, HH), lambda i: (1, i, 0)),
            pl.BlockSpec((H, H), lambda i: (0, 0)),
            pl.BlockSpec((1, H), lambda i: (0, 0)),
            pl.BlockSpec((1, H), lambda i: (0, 0)),
            pl.BlockSpec((1, H), lambda i: (0, 0)),
        ],
        out_specs=pl.BlockSpec((2000, H), lambda i: (i, 0)),
        out_shape=jax.ShapeDtypeStruct((N, H), jnp.float32),
    )(h_pad, agg, agg, W_upd, b_upd, g2, b2)


def _sc_edge_body(a_hbm, b_hbm, c_hbm, src_hbm, dst_hbm, out_hbm,
                  sidx, bidx, lidx, row_a, row_b, row_c, mbuf, agg_sh,
                  sem_a, sem_b, sem_c):
    cid = lax.axis_index("c")
    sid = lax.axis_index("s")
    rbase = cid * NP
    soff = sid * STRIPE
    ebase = sid * TILE_EP
    cbase4 = cid * (E_PAD // 4) + sid * (TILE_EP // 4)

    zero16 = jnp.zeros((16,), jnp.float32)

    def _zb(i, _):
        mbuf[i, pl.ds(0, 16)] = zero16
        mbuf[i, pl.ds(16, 16)] = zero16
        return 0

    lax.fori_loop(0, SB, _zb, 0)
    for k in range(STRIPE // SB):
        pltpu.sync_copy(mbuf, agg_sh.at[pl.ds(soff + k * SB, SB)])
    pltpu.sync_copy(mbuf.at[pl.ds(0, STRIPE % SB)],
                    agg_sh.at[pl.ds(soff + (STRIPE // SB) * SB, STRIPE % SB)])
    plsc.subcore_barrier()

    def _batch(t, _):
        off = ebase + t * SB
        pltpu.sync_copy(src_hbm.at[pl.ds(off, SB)], sidx)
        pltpu.sync_copy(dst_hbm.at[pl.ds(off, SB)], lidx)
        for k in range(SB // 16):
            sl = pl.ds(k * 16, 16)
            sidx[sl] = sidx[sl] + rbase
        cp_a = pltpu.async_copy(a_hbm.at[sidx], row_a, sem_a)
        cp_c = pltpu.async_copy(c_hbm.at[pl.ds(cbase4 + t * (SB // 4), SB // 4)],
                                row_c, sem_c)
        for k in range(SB // 16):
            sl = pl.ds(k * 16, 16)
            bidx[sl] = lidx[sl] + rbase
        cp_b = pltpu.async_copy(b_hbm.at[bidx], row_b, sem_b)
        cp_a.wait()
        cp_b.wait()
        cp_c.wait()

        for i in range(SB):
            for j in range(HH // 16):
                sl = pl.ds(j * 16, 16)
                s = (row_a[i, sl] + row_b[i, sl]
                     + row_c[i // 4, pl.ds((i % 4) * HH + j * 16, 16)])
                inner = s * (_K1 + _K2 * s * s)
                mbuf[i, pl.ds(j * 16, 16)] = s / (1.0 + jnp.exp(inner))
        pltpu.sync_copy(mbuf, agg_sh.at[lidx], add=True)
        return 0

    lax.fori_loop(0, NSB, _batch, 0)

    plsc.subcore_barrier()
    pltpu.sync_copy(agg_sh.at[pl.ds(soff, STRIPE)],
                    out_hbm.at[cid, pl.ds(soff, STRIPE)])


def _sc_edge(A, B, C4, src_pad, dst_pad):
    mesh = plsc.VectorSubcoreMesh(core_axis_name="c", subcore_axis_name="s")
    fn = pl.kernel(
        _sc_edge_body, mesh=mesh,
        out_type=jax.ShapeDtypeStruct((2, NP, HH), jnp.float32),
        compiler_params=pltpu.CompilerParams(use_tc_tiling_on_sc=False),
        scratch_types=[
            pltpu.VMEM((SB,), jnp.int32),
            pltpu.VMEM((SB,), jnp.int32),
            pltpu.VMEM((SB,), jnp.int32),
            pltpu.VMEM((SB, HH), jnp.float32),
            pltpu.VMEM((SB, HH), jnp.float32),
            pltpu.VMEM((SB // 4, 4 * HH), jnp.float32),
            pltpu.VMEM((SB, HH), jnp.float32),
            pltpu.VMEM_SHARED((NP, HH), jnp.float32),
            pltpu.SemaphoreType.DMA,
            pltpu.SemaphoreType.DMA,
            pltpu.SemaphoreType.DMA,
        ],
    )
    return fn(A.reshape(2 * NP, HH), B.reshape(2 * NP, HH),
              C4.reshape(2 * (E_PAD // 4), 4 * HH), src_pad, dst_pad)


def kernel(x, edge_index, edge_attr, W_in, b_in, ln1_g, ln1_b,
           W_msg, b_msg, W_upd, b_upd, ln2_g, ln2_b):
    W1 = W_msg[:H]
    W2 = W_msg[H:2 * H]
    W3 = W_msg[2 * H:]
    x_pad = jnp.pad(x, ((0, NP - N), (0, 0)))
    ea4 = jnp.pad(edge_attr.astype(jnp.float32),
                  ((0, E_PAD - E), (0, 0))).reshape(E_PAD // 4, 4 * EDGE_DIM)
    src_pad = jnp.pad(edge_index[0], (0, E_PAD - E))
    dst_pad = jnp.pad(edge_index[1], (0, E_PAD - E), constant_values=N)
    eye4 = jnp.eye(4, dtype=jnp.float32)
    W3_0 = jnp.kron(eye4, W3[:, :HH])
    W3_1 = jnp.kron(eye4, W3[:, HH:])
    b4_0 = jnp.tile(b_msg[:HH], 4).reshape(1, 4 * HH)
    b4_1 = jnp.tile(b_msg[HH:], 4).reshape(1, 4 * HH)
    h_pad, A, B = _tc_pre(x_pad, W_in, b_in.reshape(1, H),
                          ln1_g.reshape(1, H), ln1_b.reshape(1, H), W1, W2)
    C4 = _tc_cproj(ea4, W3_0, W3_1, b4_0, b4_1)
    agg = _sc_edge(A, B, C4, src_pad, dst_pad)
    return _tc_post(h_pad, agg, W_upd, b_upd.reshape(1, H),
                    ln2_g.reshape(1, H), ln2_b.reshape(1, H))


# speedup vs baseline: 4.2707x; 2.2970x over previous
import jax
import jax.numpy as jnp
from jax import lax
from jax.experimental import pallas as pl
from jax.experimental.pallas import tpu as pltpu
from jax.experimental.pallas import tpu_sc as plsc

N = 50000
E = 800000
NODE_DIM = 7
EDGE_DIM = 5
H = 64
HH = H // 2

NP = 50048
STRIPE = NP // 16
TILE_EP = NP
E_PAD = 16 * TILE_EP
SB = 128
NSB = TILE_EP // SB

_C = 0.7978845608028654
_K1 = -2.0 * _C
_K2 = -2.0 * _C * 0.044715


def _ln(x, g, b):
    m = jnp.mean(x, axis=-1, keepdims=True)
    v = jnp.mean((x - m) ** 2, axis=-1, keepdims=True)
    return (x - m) * jax.lax.rsqrt(v + 1e-5) * g + b


_NODE_BLK = 3128
_EDGE_BLK4 = 3128


def _pre_body(x_ref, w_in_ref, b_in_ref, g1_ref, b1_ref, w1_ref, w2_ref,
              h_ref, a_ref, b_ref):
    h = jnp.dot(x_ref[...], w_in_ref[...], preferred_element_type=jnp.float32)
    h = h + b_in_ref[...]
    h = _ln(h, g1_ref[...], b1_ref[...])
    h = 0.5 * h * (1.0 + lax.erf(h * 0.7071067811865476))
    h_ref[...] = h
    a = jnp.dot(h, w1_ref[...], preferred_element_type=jnp.float32)
    b = jnp.dot(h, w2_ref[...], preferred_element_type=jnp.float32)
    a_ref[0] = a[:, :HH]
    a_ref[1] = a[:, HH:]
    b_ref[0] = b[:, :HH]
    b_ref[1] = b[:, HH:]


def _tc_pre(x_pad, W_in, b_in, g1, b1, W1, W2):
    grid = (NP // _NODE_BLK,)
    return pl.pallas_call(
        _pre_body,
        grid=grid,
        in_specs=[
            pl.BlockSpec((_NODE_BLK, NODE_DIM), lambda i: (i, 0)),
            pl.BlockSpec((NODE_DIM, H), lambda i: (0, 0)),
            pl.BlockSpec((1, H), lambda i: (0, 0)),
            pl.BlockSpec((1, H), lambda i: (0, 0)),
            pl.BlockSpec((1, H), lambda i: (0, 0)),
            pl.BlockSpec((H, H), lambda i: (0, 0)),
            pl.BlockSpec((H, H), lambda i: (0, 0)),
        ],
        out_specs=[
            pl.BlockSpec((_NODE_BLK, H), lambda i: (i, 0)),
            pl.BlockSpec((2, _NODE_BLK, HH), lambda i: (0, i, 0)),
            pl.BlockSpec((2, _NODE_BLK, HH), lambda i: (0, i, 0)),
        ],
        out_shape=[
            jax.ShapeDtypeStruct((NP, H), jnp.float32),
            jax.ShapeDtypeStruct((2, NP, HH), jnp.float32),
            jax.ShapeDtypeStruct((2, NP, HH), jnp.float32),
        ],
    )(x_pad, W_in, b_in, g1, b1, W1, W2)


def _cproj_body(ea4_ref, w0_ref, w1_ref, b0_ref, b1_ref, c_ref):
    ea4 = ea4_ref[...]
    c_ref[0] = jnp.dot(ea4, w0_ref[...],
                       preferred_element_type=jnp.float32) + b0_ref[...]
    c_ref[1] = jnp.dot(ea4, w1_ref[...],
                       preferred_element_type=jnp.float32) + b1_ref[...]


def _tc_cproj(ea4, W3_0, W3_1, b4_0, b4_1):
    grid = (E_PAD // 4 // _EDGE_BLK4,)
    return pl.pallas_call(
        _cproj_body,
        grid=grid,
        in_specs=[
            pl.BlockSpec((_EDGE_BLK4, 4 * EDGE_DIM), lambda i: (i, 0)),
            pl.BlockSpec((4 * EDGE_DIM, 4 * HH), lambda i: (0, 0)),
            pl.BlockSpec((4 * EDGE_DIM, 4 * HH), lambda i: (0, 0)),
            pl.BlockSpec((1, 4 * HH), lambda i: (0, 0)),
            pl.BlockSpec((1, 4 * HH), lambda i: (0, 0)),
        ],
        out_specs=pl.BlockSpec((2, _EDGE_BLK4, 4 * HH), lambda i: (0, i, 0)),
        out_shape=jax.ShapeDtypeStruct((2, E_PAD // 4, 4 * HH), jnp.float32),
    )(ea4, W3_0, W3_1, b4_0, b4_1)


def _post_body(h_ref, g0_ref, g1_ref, wu_ref, bu_ref, ln_g_ref, ln_b_ref,
               o_ref):
    wu = wu_ref[...]
    upd = jnp.dot(g0_ref[0], wu[:HH, :], preferred_element_type=jnp.float32)
    upd = upd + jnp.dot(g1_ref[0], wu[HH:, :],
                        preferred_element_type=jnp.float32)
    upd = upd + bu_ref[...]
    o_ref[...] = _ln(h_ref[...] + upd, ln_g_ref[...], ln_b_ref[...])


def _tc_post(h_pad, agg, W_upd, b_upd, g2, b2):
    grid = (N // 2000,)
    return pl.pallas_call(
        _post_body,
        grid=grid,
        in_specs=[
            pl.BlockSpec((2000, H), lambda i: (i, 0)),
            pl.BlockSpec((1, 2000, HH), lambda i: (0, i, 0)),
            pl.BlockSpec((1, 2000, HH), lambda i: (1, i, 0)),
            pl.BlockSpec((H, H), lambda i: (0, 0)),
            pl.BlockSpec((1, H), lambda i: (0, 0)),
            pl.BlockSpec((1, H), lambda i: (0, 0)),
            pl.BlockSpec((1, H), lambda i: (0, 0)),
        ],
        out_specs=pl.BlockSpec((2000, H), lambda i: (i, 0)),
        out_shape=jax.ShapeDtypeStruct((N, H), jnp.float32),
    )(h_pad, agg, agg, W_upd, b_upd, g2, b2)


_GC = (3.9828342073e-01, -6.5184417508e-02, 9.0751180176e-03,
       -8.8165705090e-04, 5.5490604605e-05, -2.0070767524e-06,
       3.1365727279e-08)


def _gelu_poly(s):
    t = jnp.minimum(jnp.maximum(s, -3.8), 3.8)
    u = t * t
    p = jnp.float32(_GC[6])
    for c in _GC[5::-1]:
        p = p * u + jnp.float32(c)
    return s * (0.5 + t * p)


def _sc_edge_body(a_hbm, b_hbm, c_hbm, src_hbm, dst_hbm, out_hbm,
                  sidx0, bidx0, lidx0, sidx1, bidx1, lidx1,
                  ra0, rb0, rc0, ra1, rb1, rc1, mb0, agg_sh,
                  si0, sd0, sa0, sba0, sca0, si1, sd1, sa1, sba1, sca1):
    cid = lax.axis_index("c")
    sid = lax.axis_index("s")
    rbase = cid * NP
    soff = sid * STRIPE
    ebase = sid * TILE_EP
    cbase4 = cid * (E_PAD // 4) + sid * (TILE_EP // 4)

    zero16 = jnp.zeros((16,), jnp.float32)

    def _zb(i, _):
        mb0[i, pl.ds(0, 16)] = zero16
        mb0[i, pl.ds(16, 16)] = zero16
        return 0

    lax.fori_loop(0, SB, _zb, 0)
    for k in range(STRIPE // SB):
        pltpu.sync_copy(mb0, agg_sh.at[pl.ds(soff + k * SB, SB)])
    pltpu.sync_copy(mb0.at[pl.ds(0, STRIPE % SB)],
                    agg_sh.at[pl.ds(soff + (STRIPE // SB) * SB, STRIPE % SB)])
    plsc.subcore_barrier()

    slot0 = (sidx0, bidx0, lidx0, ra0, rb0, rc0, mb0, si0, sd0, sa0, sba0, sca0)
    slot1 = (sidx1, bidx1, lidx1, ra1, rb1, rc1, mb0, si1, sd1, sa1, sba1, sca1)
    last = NSB - 1

    def s1(sl, t):
        sidx, bidx, lidx, ra, rb, rc, mb, s_i, s_d, sa, sbm, scm = sl
        off = pl.multiple_of(ebase + t * SB, 128)
        pltpu.async_copy(src_hbm.at[pl.ds(off, SB)], sidx, s_i)
        pltpu.async_copy(dst_hbm.at[pl.ds(off, SB)], lidx, s_d)

    def s2(sl, t):
        sidx, bidx, lidx, ra, rb, rc, mb, s_i, s_d, sa, sbm, scm = sl
        pltpu.make_async_copy(src_hbm.at[pl.ds(0, SB)], sidx, s_i).wait()
        pltpu.make_async_copy(dst_hbm.at[pl.ds(0, SB)], lidx, s_d).wait()
        for k in range(SB // 16):
            ssl = pl.ds(k * 16, 16)
            sidx[ssl] = sidx[ssl] + rbase
            bidx[ssl] = lidx[ssl] + rbase
        pltpu.async_copy(a_hbm.at[sidx], ra, sa)
        pltpu.async_copy(b_hbm.at[bidx], rb, sbm)
        pltpu.async_copy(c_hbm.at[pl.ds(cbase4 + t * (SB // 4), SB // 4)],
                         rc, scm)

    def s3(sl):
        sidx, bidx, lidx, ra, rb, rc, mb, s_i, s_d, sa, sbm, scm = sl
        pltpu.make_async_copy(a_hbm.at[pl.ds(0, SB)], ra, sa).wait()
        pltpu.make_async_copy(b_hbm.at[pl.ds(0, SB)], rb, sbm).wait()
        pltpu.make_async_copy(c_hbm.at[pl.ds(0, SB // 4)], rc, scm).wait()

        def _g4(g4, _):
            for q in range(4):
                for j in range(HH // 16):
                    sl16 = pl.ds(j * 16, 16)
                    s = (ra[g4 * 4 + q, sl16] + rb[g4 * 4 + q, sl16]
                         + rc[g4, pl.ds(q * HH + j * 16, 16)])
                    mb[g4 * 4 + q, sl16] = _gelu_poly(s)
            return 0

        lax.fori_loop(0, SB // 4, _g4, 0)
        pltpu.sync_copy(mb, agg_sh.at[lidx], add=True)

    s1(slot0, 0)
    s2(slot0, 0)
    s1(slot1, 1)

    def _pair(g, _):
        s2(slot1, 2 * g + 1)
        s3(slot0)
        s1(slot0, jnp.minimum(2 * g + 2, last))
        s2(slot0, 2 * g + 2)
        s3(slot1)
        s1(slot1, jnp.minimum(2 * g + 3, last))
        return 0

    lax.fori_loop(0, (NSB - 1) // 2, _pair, 0)
    s3(slot0)
    pltpu.make_async_copy(src_hbm.at[pl.ds(0, SB)], sidx1, si1).wait()
    pltpu.make_async_copy(dst_hbm.at[pl.ds(0, SB)], lidx1, sd1).wait()

    plsc.subcore_barrier()
    pltpu.sync_copy(agg_sh.at[pl.ds(soff, STRIPE)],
                    out_hbm.at[cid, pl.ds(soff, STRIPE)])


def _sc_edge(A, B, C4, src_pad, dst_pad):
    mesh = plsc.VectorSubcoreMesh(core_axis_name="c", subcore_axis_name="s")
    fn = pl.kernel(
        _sc_edge_body, mesh=mesh,
        out_type=jax.ShapeDtypeStruct((2, NP, HH), jnp.float32),
        compiler_params=pltpu.CompilerParams(use_tc_tiling_on_sc=False),
        scratch_types=(
            [pltpu.VMEM((SB,), jnp.int32)] * 6
            + [pltpu.VMEM((SB, HH), jnp.float32),
               pltpu.VMEM((SB, HH), jnp.float32),
               pltpu.VMEM((SB // 4, 4 * HH), jnp.float32),
               pltpu.VMEM((SB, HH), jnp.float32),
               pltpu.VMEM((SB, HH), jnp.float32),
               pltpu.VMEM((SB // 4, 4 * HH), jnp.float32),
               pltpu.VMEM((SB, HH), jnp.float32),
               pltpu.VMEM_SHARED((NP, HH), jnp.float32)]
            + [pltpu.SemaphoreType.DMA] * 10
        ),
    )
    return fn(A.reshape(2 * NP, HH), B.reshape(2 * NP, HH),
              C4.reshape(2 * (E_PAD // 4), 4 * HH), src_pad, dst_pad)


def kernel(x, edge_index, edge_attr, W_in, b_in, ln1_g, ln1_b,
           W_msg, b_msg, W_upd, b_upd, ln2_g, ln2_b):
    W1 = W_msg[:H]
    W2 = W_msg[H:2 * H]
    W3 = W_msg[2 * H:]
    x_pad = jnp.pad(x, ((0, NP - N), (0, 0)))
    ea4 = jnp.pad(edge_attr.astype(jnp.float32),
                  ((0, E_PAD - E), (0, 0))).reshape(E_PAD // 4, 4 * EDGE_DIM)
    src_pad = jnp.pad(edge_index[0], (0, E_PAD - E))
    dst_pad = jnp.pad(edge_index[1], (0, E_PAD - E), constant_values=N)
    eye4 = jnp.eye(4, dtype=jnp.float32)
    W3_0 = jnp.kron(eye4, W3[:, :HH])
    W3_1 = jnp.kron(eye4, W3[:, HH:])
    b4_0 = jnp.tile(b_msg[:HH], 4).reshape(1, 4 * HH)
    b4_1 = jnp.tile(b_msg[HH:], 4).reshape(1, 4 * HH)
    h_pad, A, B = _tc_pre(x_pad, W_in, b_in.reshape(1, H),
                          ln1_g.reshape(1, H), ln1_b.reshape(1, H), W1, W2)
    C4 = _tc_cproj(ea4, W3_0, W3_1, b4_0, b4_1)
    agg = _sc_edge(A, B, C4, src_pad, dst_pad)
    return _tc_post(h_pad, agg, W_upd, b_upd.reshape(1, H),
                    ln2_g.reshape(1, H), ln2_b.reshape(1, H))


# speedup vs baseline: 4.9952x; 1.1696x over previous
import jax
import jax.numpy as jnp
from jax import lax
from jax.experimental import pallas as pl
from jax.experimental.pallas import tpu as pltpu
from jax.experimental.pallas import tpu_sc as plsc

N = 50000
E = 800000
NODE_DIM = 7
EDGE_DIM = 5
H = 64
HH = H // 2

NP = 50048
STRIPE = NP // 16
TILE_EP = NP
E_PAD = 16 * TILE_EP
SB = 128
NSB = TILE_EP // SB

_C = 0.7978845608028654
_K1 = -2.0 * _C
_K2 = -2.0 * _C * 0.044715


def _ln(x, g, b):
    m = jnp.mean(x, axis=-1, keepdims=True)
    v = jnp.mean((x - m) ** 2, axis=-1, keepdims=True)
    return (x - m) * jax.lax.rsqrt(v + 1e-5) * g + b


_NODE_BLK = 3128


def _pre_body(x_ref, w_in_ref, b_in_ref, g1_ref, b1_ref, w1_ref, w2_ref,
              h_ref, a_ref, b_ref):
    h = jnp.dot(x_ref[...], w_in_ref[...], preferred_element_type=jnp.float32)
    h = h + b_in_ref[...]
    h = _ln(h, g1_ref[...], b1_ref[...])
    h = 0.5 * h * (1.0 + lax.erf(h * 0.7071067811865476))
    h_ref[...] = h
    a = jnp.dot(h, w1_ref[...], preferred_element_type=jnp.float32)
    b = jnp.dot(h, w2_ref[...], preferred_element_type=jnp.float32)
    a_ref[0] = a[:, :HH]
    a_ref[1] = a[:, HH:]
    b_ref[0] = b[:, :HH]
    b_ref[1] = b[:, HH:]


def _tc_pre(x_pad, W_in, b_in, g1, b1, W1, W2):
    grid = (NP // _NODE_BLK,)
    return pl.pallas_call(
        _pre_body,
        grid=grid,
        in_specs=[
            pl.BlockSpec((_NODE_BLK, NODE_DIM), lambda i: (i, 0)),
            pl.BlockSpec((NODE_DIM, H), lambda i: (0, 0)),
            pl.BlockSpec((1, H), lambda i: (0, 0)),
            pl.BlockSpec((1, H), lambda i: (0, 0)),
            pl.BlockSpec((1, H), lambda i: (0, 0)),
            pl.BlockSpec((H, H), lambda i: (0, 0)),
            pl.BlockSpec((H, H), lambda i: (0, 0)),
        ],
        out_specs=[
            pl.BlockSpec((_NODE_BLK, H), lambda i: (i, 0)),
            pl.BlockSpec((2, _NODE_BLK, HH), lambda i: (0, i, 0)),
            pl.BlockSpec((2, _NODE_BLK, HH), lambda i: (0, i, 0)),
        ],
        out_shape=[
            jax.ShapeDtypeStruct((NP, H), jnp.float32),
            jax.ShapeDtypeStruct((2, NP, HH), jnp.float32),
            jax.ShapeDtypeStruct((2, NP, HH), jnp.float32),
        ],
    )(x_pad, W_in, b_in, g1, b1, W1, W2)


def _cproj_body(ea4_ref, w0_ref, w1_ref, b0_ref, b1_ref, c_ref):
    ea4 = ea4_ref[...]
    c_ref[0] = jnp.dot(ea4, w0_ref[...],
                       preferred_element_type=jnp.float32) + b0_ref[...]
    c_ref[1] = jnp.dot(ea4, w1_ref[...],
                       preferred_element_type=jnp.float32) + b1_ref[...]


_EBLK4 = 2000


def _tc_cproj(ea4, W3_0, W3_1, b4_0, b4_1):
    grid = (E // 4 // _EBLK4,)
    return pl.pallas_call(
        _cproj_body,
        grid=grid,
        in_specs=[
            pl.BlockSpec((_EBLK4, 4 * EDGE_DIM), lambda i: (i, 0)),
            pl.BlockSpec((4 * EDGE_DIM, 4 * HH), lambda i: (0, 0)),
            pl.BlockSpec((4 * EDGE_DIM, 4 * HH), lambda i: (0, 0)),
            pl.BlockSpec((1, 4 * HH), lambda i: (0, 0)),
            pl.BlockSpec((1, 4 * HH), lambda i: (0, 0)),
        ],
        out_specs=pl.BlockSpec((2, _EBLK4, 4 * HH), lambda i: (0, i, 0)),
        out_shape=jax.ShapeDtypeStruct((2, E_PAD // 4, 4 * HH), jnp.float32),
    )(ea4, W3_0, W3_1, b4_0, b4_1)


def _post_body(h_ref, g0_ref, g1_ref, wu_ref, bu_ref, ln_g_ref, ln_b_ref,
               o_ref):
    wu = wu_ref[...]
    upd = jnp.dot(g0_ref[0], wu[:HH, :], preferred_element_type=jnp.float32)
    upd = upd + jnp.dot(g1_ref[0], wu[HH:, :],
                        preferred_element_type=jnp.float32)
    upd = upd + bu_ref[...]
    o_ref[...] = _ln(h_ref[...] + upd, ln_g_ref[...], ln_b_ref[...])


def _tc_post(h_pad, agg, W_upd, b_upd, g2, b2):
    grid = (N // 2000,)
    return pl.pallas_call(
        _post_body,
        grid=grid,
        in_specs=[
            pl.BlockSpec((2000, H), lambda i: (i, 0)),
            pl.BlockSpec((1, 2000, HH), lambda i: (0, i, 0)),
            pl.BlockSpec((1, 2000, HH), lambda i: (1, i, 0)),
            pl.BlockSpec((H, H), lambda i: (0, 0)),
            pl.BlockSpec((1, H), lambda i: (0, 0)),
            pl.BlockSpec((1, H), lambda i: (0, 0)),
            pl.BlockSpec((1, H), lambda i: (0, 0)),
        ],
        out_specs=pl.BlockSpec((2000, H), lambda i: (i, 0)),
        out_shape=jax.ShapeDtypeStruct((N, H), jnp.float32),
    )(h_pad, agg, agg, W_upd, b_upd, g2, b2)


_GC = (3.9828342073e-01, -6.5184417508e-02, 9.0751180176e-03,
       -8.8165705090e-04, 5.5490604605e-05, -2.0070767524e-06,
       3.1365727279e-08)


def _gelu_poly(s):
    t = jnp.minimum(jnp.maximum(s, -3.8), 3.8)
    u = t * t
    p = jnp.float32(_GC[6])
    for c in _GC[5::-1]:
        p = p * u + jnp.float32(c)
    return s * (0.5 + t * p)


def _sc_edge_body(a_hbm, b_hbm, c_hbm, src_hbm, dst_hbm, out_hbm,
                  sidx0, bidx0, lidx0, sidx1, bidx1, lidx1,
                  ra0, rb0, rc0, ra1, rb1, rc1, mb0, agg_sh,
                  si0, sd0, sa0, sba0, sca0, si1, sd1, sa1, sba1, sca1):
    cid = lax.axis_index("c")
    sid = lax.axis_index("s")
    rbase = cid * NP
    soff = sid * STRIPE
    ebase = sid * TILE_EP
    cbase4 = cid * (E_PAD // 4) + sid * (TILE_EP // 4)

    zero16 = jnp.zeros((16,), jnp.float32)

    def _zb(i, _):
        mb0[i, pl.ds(0, 16)] = zero16
        mb0[i, pl.ds(16, 16)] = zero16
        return 0

    lax.fori_loop(0, SB, _zb, 0)
    for k in range(STRIPE // SB):
        pltpu.sync_copy(mb0, agg_sh.at[pl.ds(soff + k * SB, SB)])
    pltpu.sync_copy(mb0.at[pl.ds(0, STRIPE % SB)],
                    agg_sh.at[pl.ds(soff + (STRIPE // SB) * SB, STRIPE % SB)])
    plsc.subcore_barrier()

    slot0 = (sidx0, bidx0, lidx0, ra0, rb0, rc0, mb0, si0, sd0, sa0, sba0, sca0)
    slot1 = (sidx1, bidx1, lidx1, ra1, rb1, rc1, mb0, si1, sd1, sa1, sba1, sca1)
    last = NSB - 1

    def s1(sl, t):
        sidx, bidx, lidx, ra, rb, rc, mb, s_i, s_d, sa, sbm, scm = sl
        off = pl.multiple_of(ebase + t * SB, 128)
        pltpu.async_copy(src_hbm.at[pl.ds(off, SB)], sidx, s_i)
        pltpu.async_copy(dst_hbm.at[pl.ds(off, SB)], lidx, s_d)

    def s2(sl, t):
        sidx, bidx, lidx, ra, rb, rc, mb, s_i, s_d, sa, sbm, scm = sl
        pltpu.make_async_copy(src_hbm.at[pl.ds(0, SB)], sidx, s_i).wait()
        pltpu.make_async_copy(dst_hbm.at[pl.ds(0, SB)], lidx, s_d).wait()
        for k in range(SB // 16):
            ssl = pl.ds(k * 16, 16)
            sidx[ssl] = sidx[ssl] + rbase
            bidx[ssl] = lidx[ssl] + rbase
        pltpu.async_copy(a_hbm.at[sidx], ra, sa)
        pltpu.async_copy(b_hbm.at[bidx], rb, sbm)
        pltpu.async_copy(c_hbm.at[pl.ds(cbase4 + t * (SB // 4), SB // 4)],
                         rc, scm)

    def s3(sl):
        sidx, bidx, lidx, ra, rb, rc, mb, s_i, s_d, sa, sbm, scm = sl
        pltpu.make_async_copy(a_hbm.at[pl.ds(0, SB)], ra, sa).wait()
        pltpu.make_async_copy(b_hbm.at[pl.ds(0, SB)], rb, sbm).wait()
        pltpu.make_async_copy(c_hbm.at[pl.ds(0, SB // 4)], rc, scm).wait()

        def _g4(g4, _):
            for q in range(4):
                for j in range(HH // 16):
                    sl16 = pl.ds(j * 16, 16)
                    s = (ra[g4 * 4 + q, sl16] + rb[g4 * 4 + q, sl16]
                         + rc[g4, pl.ds(q * HH + j * 16, 16)])
                    mb[g4 * 4 + q, sl16] = _gelu_poly(s)
            return 0

        lax.fori_loop(0, SB // 4, _g4, 0)
        pltpu.sync_copy(mb, agg_sh.at[lidx], add=True)

    s1(slot0, 0)
    s2(slot0, 0)
    s1(slot1, 1)

    def _pair(g, _):
        s2(slot1, 2 * g + 1)
        s3(slot0)
        s1(slot0, jnp.minimum(2 * g + 2, last))
        s2(slot0, 2 * g + 2)
        s3(slot1)
        s1(slot1, jnp.minimum(2 * g + 3, last))
        return 0

    lax.fori_loop(0, (NSB - 1) // 2, _pair, 0)
    s3(slot0)
    pltpu.make_async_copy(src_hbm.at[pl.ds(0, SB)], sidx1, si1).wait()
    pltpu.make_async_copy(dst_hbm.at[pl.ds(0, SB)], lidx1, sd1).wait()

    plsc.subcore_barrier()
    pltpu.sync_copy(agg_sh.at[pl.ds(soff, STRIPE)],
                    out_hbm.at[cid, pl.ds(soff, STRIPE)])


def _sc_edge(A, B, C4, src_pad, dst_pad):
    mesh = plsc.VectorSubcoreMesh(core_axis_name="c", subcore_axis_name="s")
    fn = pl.kernel(
        _sc_edge_body, mesh=mesh,
        out_type=jax.ShapeDtypeStruct((2, NP, HH), jnp.float32),
        compiler_params=pltpu.CompilerParams(use_tc_tiling_on_sc=False),
        scratch_types=(
            [pltpu.VMEM((SB,), jnp.int32)] * 6
            + [pltpu.VMEM((SB, HH), jnp.float32),
               pltpu.VMEM((SB, HH), jnp.float32),
               pltpu.VMEM((SB // 4, 4 * HH), jnp.float32),
               pltpu.VMEM((SB, HH), jnp.float32),
               pltpu.VMEM((SB, HH), jnp.float32),
               pltpu.VMEM((SB // 4, 4 * HH), jnp.float32),
               pltpu.VMEM((SB, HH), jnp.float32),
               pltpu.VMEM_SHARED((NP, HH), jnp.float32)]
            + [pltpu.SemaphoreType.DMA] * 10
        ),
    )
    return fn(A.reshape(2 * NP, HH), B.reshape(2 * NP, HH),
              C4.reshape(2 * (E_PAD // 4), 4 * HH), src_pad, dst_pad)


def kernel(x, edge_index, edge_attr, W_in, b_in, ln1_g, ln1_b,
           W_msg, b_msg, W_upd, b_upd, ln2_g, ln2_b):
    W1 = W_msg[:H]
    W2 = W_msg[H:2 * H]
    W3 = W_msg[2 * H:]
    x_pad = jnp.pad(x, ((0, NP - N), (0, 0)))
    src_pad = jnp.pad(edge_index[0], (0, E_PAD - E))
    dst_pad = jnp.pad(edge_index[1], (0, E_PAD - E), constant_values=N)
    ea4 = edge_attr.astype(jnp.float32).reshape(E // 4, 4 * EDGE_DIM)
    eye4 = jnp.eye(4, dtype=jnp.float32)
    W3_0 = jnp.kron(eye4, W3[:, :HH])
    W3_1 = jnp.kron(eye4, W3[:, HH:])
    b4_0 = jnp.tile(b_msg[:HH], 4).reshape(1, 4 * HH)
    b4_1 = jnp.tile(b_msg[HH:], 4).reshape(1, 4 * HH)
    h_pad, A, B = _tc_pre(x_pad, W_in, b_in.reshape(1, H),
                          ln1_g.reshape(1, H), ln1_b.reshape(1, H), W1, W2)
    C4 = _tc_cproj(ea4, W3_0, W3_1, b4_0, b4_1)
    agg = _sc_edge(A, B, C4, src_pad, dst_pad)
    return _tc_post(h_pad, agg, W_upd, b_upd.reshape(1, H),
                    ln2_g.reshape(1, H), ln2_b.reshape(1, H))
